# baseline (speedup 1.0000x reference)
import math
import functools

import jax
import jax.numpy as jnp
import numpy as np
from jax import lax
from jax.experimental import pallas as pl
from jax.experimental.pallas import tpu as pltpu
from jax.experimental.pallas import tpu_sc as plsc

_ALPHA = 2.0
_NUM_BINS = 20
_THRESH_FOD = 0.05
_THRESH_MASK = 0.3

_RGB_FROM_HED = np.array(
    [[0.65, 0.7, 0.29], [0.07, 0.99, 0.11], [0.27, 0.57, 0.78]], dtype=np.float64
)
_HED_FROM_RGB = np.linalg.inv(_RGB_FROM_HED)
_LOG_ADJUST = math.log(1e-6)
_ADJ_CAL = float(10.0 ** (-(math.e ** (1.0 / _ALPHA))))
_COEFFS = (0.2125, 0.7154, 0.0721)
_INV_LN10 = 1.0 / math.log(10.0)
_BIN_SCALE = _NUM_BINS / math.e


def _to_bf16_f32(v):
    import ml_dtypes

    return float(np.asarray(v, np.float32).astype(ml_dtypes.bfloat16).astype(np.float32))


def _branch_consts(idx):
    col = tuple(_to_bf16_f32(_HED_FROM_RGB[j, idx]) for j in range(3))
    row = tuple(_to_bf16_f32(_RGB_FROM_HED[idx, j]) for j in range(3))
    return col, row

_COL_H, _ROW_H = _branch_consts(0)
_COL_D, _ROW_D = _branch_consts(2)
_COEFFS_BF = tuple(_to_bf16_f32(c) for c in _COEFFS)
_NEG_LOG_ADJUST = -_LOG_ADJUST

_ROWS_PER_STEP = 128
_STATS_W = 32

_SC_TILES = 32
_SC_LANES = 16
_SC_CHUNK = 16384


def _bf(x):
    return x.astype(jnp.bfloat16).astype(jnp.float32)


def _pixel_branch(lvr, lvg, lvb, col, row):
    s = col[0] * lvr + col[1] * lvg + col[2] * lvb
    s = jnp.maximum(s, 0.0)
    u = _bf(s * _NEG_LOG_ADJUST)
    grey = (
        _COEFFS_BF[0] * _bf(jnp.exp(-(u * row[0])))
        + _COEFFS_BF[1] * _bf(jnp.exp(-(u * row[1])))
        + _COEFFS_BF[2] * _bf(jnp.exp(-(u * row[2])))
    )
    grey = jnp.clip(grey, 0.0, 1.0)
    fod = jnp.log(grey + _ADJ_CAL) * (-_INV_LN10)
    fod = jnp.maximum(fod, 0.0)
    f2 = fod * fod
    fod_relu = jnp.where(f2 < _THRESH_FOD, 0.0, f2)
    mask = jnp.where(f2 < _THRESH_MASK, 0.0, 1.0)
    return f2, fod_relu, mask


def _scalar11(x):
    return jnp.reshape(x, (1, 1))


def _main_body(inp_ref, tgt_ref, mih_ref, mth_ref, mid_ref, mtd_ref, fod_ref, stats_ref):
    bb = pl.program_id(0)
    r = pl.program_id(1)

    @pl.when((bb == 0) & (r == 0))
    def _():
        stats_ref[...] = jnp.zeros_like(stats_ref)

    for x_ref, m_h_ref, m_d_ref, c_h, c_d in (
        (inp_ref, mih_ref, mid_ref, 0, 1),
        (tgt_ref, mth_ref, mtd_ref, 2, 3),
    ):
        lvr = _bf(jnp.log(jnp.maximum(x_ref[0, 0], 1e-6)) / _LOG_ADJUST)
        lvg = _bf(jnp.log(jnp.maximum(x_ref[0, 1], 1e-6)) / _LOG_ADJUST)
        lvb = _bf(jnp.log(jnp.maximum(x_ref[0, 2], 1e-6)) / _LOG_ADJUST)
        for combo, m_ref, col, row in (
            (c_h, m_h_ref, _COL_H, _ROW_H),
            (c_d, m_d_ref, _COL_D, _ROW_D),
        ):
            f2, fod_relu, mask = _pixel_branch(lvr, lvg, lvb, col, row)
            m_ref[0] = mask
            fod_ref[combo, 0] = f2
            avg = _scalar11(jnp.sum(fod_relu))
            blk = jnp.concatenate(
                [_scalar11(jnp.sum(fod_relu[:, c * 128:(c + 1) * 128])) for c in range(4)],
                axis=1,
            )
            col_iota = lax.broadcasted_iota(jnp.int32, (1, 16), 1) // 4
            blk16 = jnp.where(
                col_iota == r, jnp.concatenate([blk, blk, blk, blk], axis=1), 0.0
            )
            upd = jnp.concatenate(
                [avg, jnp.zeros((1, 7), jnp.float32), blk16,
                 jnp.zeros((1, _STATS_W - 24), jnp.float32)],
                axis=1,
            )
            row_iota = lax.broadcasted_iota(jnp.int32, (stats_ref.shape[1], 1), 0)
            stats_ref[combo] += jnp.where(row_iota == bb, 1.0, 0.0) * upd


def _sc_hist(fod_flat, rows, row_len):
    mesh = plsc.VectorSubcoreMesh(core_axis_name="c", subcore_axis_name="s")
    acc_w = _SC_LANES * _NUM_BINS
    n_chunks = row_len // _SC_CHUNK

    @functools.partial(
        pl.kernel,
        out_type=jax.ShapeDtypeStruct((rows, acc_w), jnp.float32),
        mesh=mesh,
        scratch_types=[
            pltpu.VMEM((_SC_CHUNK,), jnp.float32),
            pltpu.VMEM((acc_w,), jnp.float32),
        ],
        compiler_params=pltpu.CompilerParams(needs_layout_passes=False),
    )
    def hist_kernel(x_hbm, out_hbm, buf, acc):
        wid = lax.axis_index("s") * 2 + lax.axis_index("c")
        zero = jnp.zeros((_SC_LANES,), jnp.float32)
        for k in range(_NUM_BINS):
            acc[pl.ds(k * _SC_LANES, _SC_LANES)] = zero
        lane_base = lax.iota(jnp.int32, _SC_LANES) * _NUM_BINS
        row_off = wid * row_len

        def chunk_body(ci, _):
            pltpu.sync_copy(x_hbm.at[pl.ds(row_off + ci * _SC_CHUNK, _SC_CHUNK)], buf)

            def vec_body(i, _):
                v = buf[pl.ds(i * _SC_LANES, _SC_LANES)]
                t = jnp.minimum(v * _BIN_SCALE, float(_NUM_BINS - 1))
                fl = lane_base + t.astype(jnp.int32)
                plsc.addupdate_scatter(acc, [fl], v)
                return 0

            lax.fori_loop(0, _SC_CHUNK // _SC_LANES, vec_body, 0)
            return 0

        lax.fori_loop(0, n_chunks, chunk_body, 0)
        pltpu.sync_copy(acc, out_hbm.at[wid])

    return hist_kernel(fod_flat)


def _loss_body(stats_ref, hist_ref, out_ref, *, batch, hw):
    hist = jnp.sum(hist_ref[...], axis=2)

    def branch_loss(si, st, hi, ht):
        avg_i, avg_t = si[:, 0:1], st[:, 0:1]
        blk_i, blk_t = si[:, 8:24], st[:, 8:24]
        dcp_avg = (avg_i - avg_t) ** 2 / float(hw) ** 2
        dcp_histo = jnp.sum((hi / hw - ht / hw) ** 2, axis=1, keepdims=True) / float(batch)
        scale = 16.0 / float(hw)
        dcp_block = jnp.sum((blk_i * scale - blk_t * scale) ** 2) / float(batch * 16)
        diff = avg_i - avg_t
        cond = (diff >= avg_t * -0.4) & (diff <= avg_t * 0.4)
        return jnp.sum(jnp.where(cond, dcp_histo, dcp_avg + dcp_histo)) + dcp_block

    total = branch_loss(stats_ref[0], stats_ref[2], hist[0], hist[2]) + branch_loss(
        stats_ref[1], stats_ref[3], hist[1], hist[3]
    )
    out_ref[...] = _scalar11(total)


def kernel(inputs, targets):
    b, _, h, w = inputs.shape
    hw = h * w
    steps = h // _ROWS_PER_STEP
    grid = (b, steps)
    img_spec = pl.BlockSpec((1, 3, _ROWS_PER_STEP, w), lambda bb, rr: (bb, 0, rr, 0))
    mask_spec = pl.BlockSpec((1, _ROWS_PER_STEP, w), lambda bb, rr: (bb, rr, 0))
    fod_spec = pl.BlockSpec((4, 1, _ROWS_PER_STEP, w), lambda bb, rr: (0, bb, rr, 0))
    stats_spec = pl.BlockSpec((4, b, _STATS_W), lambda bb, rr: (0, 0, 0))

    mih, mth, mid, mtd, fod, stats = pl.pallas_call(
        _main_body,
        grid=grid,
        in_specs=[img_spec, img_spec],
        out_specs=[mask_spec, mask_spec, mask_spec, mask_spec, fod_spec, stats_spec],
        out_shape=[
            jax.ShapeDtypeStruct((b, h, w), jnp.float32),
            jax.ShapeDtypeStruct((b, h, w), jnp.float32),
            jax.ShapeDtypeStruct((b, h, w), jnp.float32),
            jax.ShapeDtypeStruct((b, h, w), jnp.float32),
            jax.ShapeDtypeStruct((4, b, h, w), jnp.float32),
            jax.ShapeDtypeStruct((4, b, _STATS_W), jnp.float32),
        ],
        compiler_params=pltpu.CompilerParams(
            dimension_semantics=("arbitrary", "arbitrary")
        ),
    )(inputs, targets)

    rows = 4 * b
    hist_part = _sc_hist(jnp.reshape(fod, (rows * hw,)), rows, hw)

    loss = pl.pallas_call(
        functools.partial(_loss_body, batch=b, hw=hw),
        out_shape=jax.ShapeDtypeStruct((1, 1), jnp.float32),
    )(stats, jnp.reshape(hist_part, (4, b, _SC_LANES, _NUM_BINS)))
    return (jnp.reshape(loss, ()), mih, mth, mid, mtd)


# speedup vs baseline: 2.0302x; 2.0302x over previous
import math
import functools

import jax
import jax.numpy as jnp
import numpy as np
from jax import lax
from jax.experimental import pallas as pl
from jax.experimental.pallas import tpu as pltpu
from jax.experimental.pallas import tpu_sc as plsc

_ALPHA = 2.0
_NUM_BINS = 20
_THRESH_FOD = 0.05
_THRESH_MASK = 0.3

_RGB_FROM_HED = np.array(
    [[0.65, 0.7, 0.29], [0.07, 0.99, 0.11], [0.27, 0.57, 0.78]], dtype=np.float64
)
_HED_FROM_RGB = np.linalg.inv(_RGB_FROM_HED)
_LOG_ADJUST = math.log(1e-6)
_ADJ_CAL = float(10.0 ** (-(math.e ** (1.0 / _ALPHA))))
_COEFFS = (0.2125, 0.7154, 0.0721)
_INV_LN10 = 1.0 / math.log(10.0)
_BIN_SCALE = _NUM_BINS / math.e


def _to_bf16_f32(v):
    import ml_dtypes

    return float(np.asarray(v, np.float32).astype(ml_dtypes.bfloat16).astype(np.float32))


def _branch_consts(idx):
    col = tuple(_to_bf16_f32(_HED_FROM_RGB[j, idx]) for j in range(3))
    row = tuple(_to_bf16_f32(_RGB_FROM_HED[idx, j]) for j in range(3))
    return col, row

_COL_H, _ROW_H = _branch_consts(0)
_COL_D, _ROW_D = _branch_consts(2)
_COEFFS_BF = tuple(_to_bf16_f32(c) for c in _COEFFS)
_NEG_LOG_ADJUST = -_LOG_ADJUST

_ROWS_PER_STEP = 128
_STATS_W = 32

_SC_TILES = 32
_SC_LANES = 16
_SC_CHUNK = 16384


def _bf(x):
    return x.astype(jnp.bfloat16).astype(jnp.float32)


def _pixel_branch(lvr, lvg, lvb, col, row):
    s = col[0] * lvr + col[1] * lvg + col[2] * lvb
    s = jnp.maximum(s, 0.0)
    u = _bf(s * _NEG_LOG_ADJUST)
    grey = (
        _COEFFS_BF[0] * _bf(jnp.exp(-(u * row[0])))
        + _COEFFS_BF[1] * _bf(jnp.exp(-(u * row[1])))
        + _COEFFS_BF[2] * _bf(jnp.exp(-(u * row[2])))
    )
    grey = jnp.clip(grey, 0.0, 1.0)
    fod = jnp.log(grey + _ADJ_CAL) * (-_INV_LN10)
    fod = jnp.maximum(fod, 0.0)
    f2 = fod * fod
    fod_relu = jnp.where(f2 < _THRESH_FOD, 0.0, f2)
    mask = jnp.where(f2 < _THRESH_MASK, 0.0, 1.0)
    return f2, fod_relu, mask


def _scalar11(x):
    return jnp.reshape(x, (1, 1))


def _main_body(inp_ref, tgt_ref, mih_ref, mth_ref, mid_ref, mtd_ref, fod_ref, stats_ref):
    bb = pl.program_id(0)
    r = pl.program_id(1)

    @pl.when((bb == 0) & (r == 0))
    def _():
        stats_ref[...] = jnp.zeros_like(stats_ref)

    for x_ref, m_h_ref, m_d_ref, c_h, c_d in (
        (inp_ref, mih_ref, mid_ref, 0, 1),
        (tgt_ref, mth_ref, mtd_ref, 2, 3),
    ):
        lvr = _bf(jnp.log(jnp.maximum(x_ref[0, 0], 1e-6)) / _LOG_ADJUST)
        lvg = _bf(jnp.log(jnp.maximum(x_ref[0, 1], 1e-6)) / _LOG_ADJUST)
        lvb = _bf(jnp.log(jnp.maximum(x_ref[0, 2], 1e-6)) / _LOG_ADJUST)
        for combo, m_ref, col, row in (
            (c_h, m_h_ref, _COL_H, _ROW_H),
            (c_d, m_d_ref, _COL_D, _ROW_D),
        ):
            f2, fod_relu, mask = _pixel_branch(lvr, lvg, lvb, col, row)
            m_ref[0] = mask
            fod_ref[combo, 0] = f2
            avg = _scalar11(jnp.sum(fod_relu))
            blk = jnp.concatenate(
                [_scalar11(jnp.sum(fod_relu[:, c * 128:(c + 1) * 128])) for c in range(4)],
                axis=1,
            )
            col_iota = lax.broadcasted_iota(jnp.int32, (1, 16), 1) // 4
            blk16 = jnp.where(
                col_iota == r, jnp.concatenate([blk, blk, blk, blk], axis=1), 0.0
            )
            upd = jnp.concatenate(
                [avg, jnp.zeros((1, 7), jnp.float32), blk16,
                 jnp.zeros((1, _STATS_W - 24), jnp.float32)],
                axis=1,
            )
            row_iota = lax.broadcasted_iota(jnp.int32, (stats_ref.shape[1], 1), 0)
            stats_ref[combo] += jnp.where(row_iota == bb, 1.0, 0.0) * upd


def _sc_hist(fod_flat, rows, row_len):
    mesh = plsc.VectorSubcoreMesh(core_axis_name="c", subcore_axis_name="s")
    acc_w = _SC_LANES * _NUM_BINS
    n_chunks = row_len // _SC_CHUNK

    @functools.partial(
        pl.kernel,
        out_type=jax.ShapeDtypeStruct((rows, acc_w), jnp.float32),
        mesh=mesh,
        scratch_types=[
            pltpu.VMEM((2, _SC_CHUNK), jnp.float32),
            pltpu.VMEM((acc_w,), jnp.float32),
            pltpu.SemaphoreType.DMA,
            pltpu.SemaphoreType.DMA,
        ],
        compiler_params=pltpu.CompilerParams(needs_layout_passes=False),
    )
    def hist_kernel(x_hbm, out_hbm, buf, acc, sem0, sem1):
        wid = lax.axis_index("s") * 2 + lax.axis_index("c")
        sems = (sem0, sem1)
        zero = jnp.zeros((_SC_LANES,), jnp.float32)
        for k in range(_NUM_BINS):
            acc[pl.ds(k * _SC_LANES, _SC_LANES)] = zero
        lane_base = lax.iota(jnp.int32, _SC_LANES) * _NUM_BINS
        row_off = wid * row_len

        def start(ci):
            return pltpu.async_copy(
                x_hbm.at[pl.ds(row_off + ci * _SC_CHUNK, _SC_CHUNK)],
                buf.at[ci % 2],
                sems[ci % 2],
            )

        descs = [None, None]
        descs[0] = start(0)
        for ci in range(n_chunks):
            bslot = ci % 2
            if ci + 1 < n_chunks:
                descs[1 - bslot] = start(ci + 1)
            descs[bslot].wait()

            @plsc.parallel_loop(0, _SC_CHUNK // _SC_LANES, 1, unroll=8)
            def vec_body(i):
                v = buf[bslot, pl.ds(i * _SC_LANES, _SC_LANES)]
                t = jnp.minimum(v * _BIN_SCALE, float(_NUM_BINS - 1))
                fl = lane_base + t.astype(jnp.int32)
                plsc.addupdate_scatter(acc, [fl], v)

        pltpu.sync_copy(acc, out_hbm.at[wid])

    return hist_kernel(fod_flat)


def _loss_body(stats_ref, hist_ref, out_ref, *, batch, hw):
    hist = jnp.sum(hist_ref[...], axis=2)

    def branch_loss(si, st, hi, ht):
        avg_i, avg_t = si[:, 0:1], st[:, 0:1]
        blk_i, blk_t = si[:, 8:24], st[:, 8:24]
        dcp_avg = (avg_i - avg_t) ** 2 / float(hw) ** 2
        dcp_histo = jnp.sum((hi / hw - ht / hw) ** 2, axis=1, keepdims=True) / float(batch)
        scale = 16.0 / float(hw)
        dcp_block = jnp.sum((blk_i * scale - blk_t * scale) ** 2) / float(batch * 16)
        diff = avg_i - avg_t
        cond = (diff >= avg_t * -0.4) & (diff <= avg_t * 0.4)
        return jnp.sum(jnp.where(cond, dcp_histo, dcp_avg + dcp_histo)) + dcp_block

    total = branch_loss(stats_ref[0], stats_ref[2], hist[0], hist[2]) + branch_loss(
        stats_ref[1], stats_ref[3], hist[1], hist[3]
    )
    out_ref[...] = _scalar11(total)


def kernel(inputs, targets):
    b, _, h, w = inputs.shape
    hw = h * w
    steps = h // _ROWS_PER_STEP
    grid = (b, steps)
    img_spec = pl.BlockSpec((1, 3, _ROWS_PER_STEP, w), lambda bb, rr: (bb, 0, rr, 0))
    mask_spec = pl.BlockSpec((1, _ROWS_PER_STEP, w), lambda bb, rr: (bb, rr, 0))
    fod_spec = pl.BlockSpec((4, 1, _ROWS_PER_STEP, w), lambda bb, rr: (0, bb, rr, 0))
    stats_spec = pl.BlockSpec((4, b, _STATS_W), lambda bb, rr: (0, 0, 0))

    mih, mth, mid, mtd, fod, stats = pl.pallas_call(
        _main_body,
        grid=grid,
        in_specs=[img_spec, img_spec],
        out_specs=[mask_spec, mask_spec, mask_spec, mask_spec, fod_spec, stats_spec],
        out_shape=[
            jax.ShapeDtypeStruct((b, h, w), jnp.float32),
            jax.ShapeDtypeStruct((b, h, w), jnp.float32),
            jax.ShapeDtypeStruct((b, h, w), jnp.float32),
            jax.ShapeDtypeStruct((b, h, w), jnp.float32),
            jax.ShapeDtypeStruct((4, b, h, w), jnp.float32),
            jax.ShapeDtypeStruct((4, b, _STATS_W), jnp.float32),
        ],
        compiler_params=pltpu.CompilerParams(
            dimension_semantics=("arbitrary", "arbitrary")
        ),
    )(inputs, targets)

    rows = 4 * b
    hist_part = _sc_hist(jnp.reshape(fod, (rows * hw,)), rows, hw)

    loss = pl.pallas_call(
        functools.partial(_loss_body, batch=b, hw=hw),
        out_shape=jax.ShapeDtypeStruct((1, 1), jnp.float32),
    )(stats, jnp.reshape(hist_part, (4, b, _SC_LANES, _NUM_BINS)))
    return (jnp.reshape(loss, ()), mih, mth, mid, mtd)


# speedup vs baseline: 2.5037x; 1.2332x over previous
import math
import functools

import jax
import jax.numpy as jnp
import numpy as np
from jax import lax
from jax.experimental import pallas as pl
from jax.experimental.pallas import tpu as pltpu
from jax.experimental.pallas import tpu_sc as plsc

_ALPHA = 2.0
_NUM_BINS = 20
_THRESH_FOD = 0.05
_THRESH_MASK = 0.3

_RGB_FROM_HED = np.array(
    [[0.65, 0.7, 0.29], [0.07, 0.99, 0.11], [0.27, 0.57, 0.78]], dtype=np.float64
)
_HED_FROM_RGB = np.linalg.inv(_RGB_FROM_HED)
_LOG_ADJUST = math.log(1e-6)
_ADJ_CAL = float(10.0 ** (-(math.e ** (1.0 / _ALPHA))))
_COEFFS = (0.2125, 0.7154, 0.0721)
_INV_LN10 = 1.0 / math.log(10.0)
_BIN_SCALE = _NUM_BINS / math.e


def _to_bf16_f32(v):
    import ml_dtypes

    return float(np.asarray(v, np.float32).astype(ml_dtypes.bfloat16).astype(np.float32))


def _branch_consts(idx):
    col = tuple(_to_bf16_f32(_HED_FROM_RGB[j, idx]) for j in range(3))
    row = tuple(_to_bf16_f32(_RGB_FROM_HED[idx, j]) for j in range(3))
    return col, row

_COL_H, _ROW_H = _branch_consts(0)
_COL_D, _ROW_D = _branch_consts(2)
_COEFFS_BF = tuple(_to_bf16_f32(c) for c in _COEFFS)
_NEG_LOG_ADJUST = -_LOG_ADJUST

_ROWS_PER_STEP = 128
_STATS_W = 32

_SC_TILES = 32
_SC_LANES = 16
_SC_CHUNK = 16384


def _bf(x):
    return x.astype(jnp.bfloat16).astype(jnp.float32)


def _pixel_branch(lvr, lvg, lvb, col, row):
    s = col[0] * lvr + col[1] * lvg + col[2] * lvb
    s = jnp.maximum(s, 0.0)
    u = _bf(s * _NEG_LOG_ADJUST)
    grey = (
        _COEFFS_BF[0] * _bf(jnp.exp(-(u * row[0])))
        + _COEFFS_BF[1] * _bf(jnp.exp(-(u * row[1])))
        + _COEFFS_BF[2] * _bf(jnp.exp(-(u * row[2])))
    )
    grey = jnp.clip(grey, 0.0, 1.0)
    fod = jnp.log(grey + _ADJ_CAL) * (-_INV_LN10)
    fod = jnp.maximum(fod, 0.0)
    f2 = fod * fod
    fod_relu = jnp.where(f2 < _THRESH_FOD, 0.0, f2)
    mask = jnp.where(f2 < _THRESH_MASK, 0.0, 1.0)
    return f2, fod_relu, mask


def _scalar11(x):
    return jnp.reshape(x, (1, 1))


def _main_body(inp_ref, tgt_ref, mih_ref, mth_ref, mid_ref, mtd_ref, fod_ref, stats_ref):
    bb = pl.program_id(0)
    cc = pl.program_id(1)

    @pl.when((bb == 0) & (cc == 0))
    def _():
        stats_ref[...] = jnp.zeros_like(stats_ref)

    for x_ref, m_h_ref, m_d_ref, c_h, c_d in (
        (inp_ref, mih_ref, mid_ref, 0, 1),
        (tgt_ref, mth_ref, mtd_ref, 2, 3),
    ):
        lvr = _bf(jnp.log(jnp.maximum(x_ref[0, 0], 1e-6)) / _LOG_ADJUST)
        lvg = _bf(jnp.log(jnp.maximum(x_ref[0, 1], 1e-6)) / _LOG_ADJUST)
        lvb = _bf(jnp.log(jnp.maximum(x_ref[0, 2], 1e-6)) / _LOG_ADJUST)
        for combo, m_ref, col, row in (
            (c_h, m_h_ref, _COL_H, _ROW_H),
            (c_d, m_d_ref, _COL_D, _ROW_D),
        ):
            f2, fod_relu, mask = _pixel_branch(lvr, lvg, lvb, col, row)
            m_ref[0] = mask
            fod_ref[0, combo, 0] = f2
            avg = _scalar11(jnp.sum(fod_relu))
            col_iota = lax.broadcasted_iota(jnp.int32, (1, 16), 1)
            blk16 = jnp.zeros((1, 16), jnp.float32)
            for rb in range(4):
                s_rb = _scalar11(jnp.sum(fod_relu[rb * 128:(rb + 1) * 128, :]))
                blk16 = blk16 + jnp.where(col_iota == rb * 4 + cc, s_rb, 0.0)
            upd = jnp.concatenate(
                [avg, jnp.zeros((1, 7), jnp.float32), blk16,
                 jnp.zeros((1, _STATS_W - 24), jnp.float32)],
                axis=1,
            )
            row_iota = lax.broadcasted_iota(jnp.int32, (stats_ref.shape[1], 1), 0)
            stats_ref[combo] += jnp.where(row_iota == bb, 1.0, 0.0) * upd


def _sc_hist(fod_flat, rows, row_len):
    mesh = plsc.VectorSubcoreMesh(core_axis_name="c", subcore_axis_name="s")
    acc_w = _SC_LANES * _NUM_BINS
    n_chunks = row_len // _SC_CHUNK

    @functools.partial(
        pl.kernel,
        out_type=jax.ShapeDtypeStruct((rows, acc_w), jnp.float32),
        mesh=mesh,
        scratch_types=[
            pltpu.VMEM((2, _SC_CHUNK), jnp.float32),
            pltpu.VMEM((acc_w,), jnp.float32),
            pltpu.SemaphoreType.DMA,
            pltpu.SemaphoreType.DMA,
        ],
        compiler_params=pltpu.CompilerParams(needs_layout_passes=False),
    )
    def hist_kernel(x_hbm, out_hbm, buf, acc, sem0, sem1):
        wid = lax.axis_index("s") * 2 + lax.axis_index("c")
        sems = (sem0, sem1)
        zero = jnp.zeros((_SC_LANES,), jnp.float32)
        for k in range(_NUM_BINS):
            acc[pl.ds(k * _SC_LANES, _SC_LANES)] = zero
        lane_base = lax.iota(jnp.int32, _SC_LANES) * _NUM_BINS
        row_off = wid * row_len

        def start(ci):
            return pltpu.async_copy(
                x_hbm.at[pl.ds(row_off + ci * _SC_CHUNK, _SC_CHUNK)],
                buf.at[ci % 2],
                sems[ci % 2],
            )

        descs = [None, None]
        descs[0] = start(0)
        for ci in range(n_chunks):
            bslot = ci % 2
            if ci + 1 < n_chunks:
                descs[1 - bslot] = start(ci + 1)
            descs[bslot].wait()

            @plsc.parallel_loop(0, _SC_CHUNK // _SC_LANES, 1, unroll=8)
            def vec_body(i):
                v = buf[bslot, pl.ds(i * _SC_LANES, _SC_LANES)]
                t = jnp.minimum(v * _BIN_SCALE, float(_NUM_BINS - 1))
                fl = lane_base + t.astype(jnp.int32)
                plsc.addupdate_scatter(acc, [fl], v)

        pltpu.sync_copy(acc, out_hbm.at[wid])

    return hist_kernel(fod_flat)


def _loss_body(stats_ref, hist_ref, out_ref, *, batch, hw):
    hist = jnp.sum(hist_ref[...], axis=2)

    def branch_loss(si, st, hi, ht):
        avg_i, avg_t = si[:, 0:1], st[:, 0:1]
        blk_i, blk_t = si[:, 8:24], st[:, 8:24]
        dcp_avg = (avg_i - avg_t) ** 2 / float(hw) ** 2
        dcp_histo = jnp.sum((hi / hw - ht / hw) ** 2, axis=1, keepdims=True) / float(batch)
        scale = 16.0 / float(hw)
        dcp_block = jnp.sum((blk_i * scale - blk_t * scale) ** 2) / float(batch * 16)
        diff = avg_i - avg_t
        cond = (diff >= avg_t * -0.4) & (diff <= avg_t * 0.4)
        return jnp.sum(jnp.where(cond, dcp_histo, dcp_avg + dcp_histo)) + dcp_block

    total = branch_loss(
        stats_ref[0], stats_ref[2], hist[:, 0, :], hist[:, 2, :]
    ) + branch_loss(stats_ref[1], stats_ref[3], hist[:, 1, :], hist[:, 3, :])
    out_ref[...] = _scalar11(total)


def kernel(inputs, targets):
    b, _, h, w = inputs.shape
    hw = h * w
    n_stripes = w // 128
    grid = (b, n_stripes)
    img_spec = pl.BlockSpec((1, 3, h, 128), lambda bb, cc: (bb, 0, 0, cc))
    mask_spec = pl.BlockSpec((1, h, 128), lambda bb, cc: (bb, 0, cc))
    fod_spec = pl.BlockSpec((1, 4, 1, h, 128), lambda bb, cc: (bb, 0, cc, 0, 0))
    stats_spec = pl.BlockSpec((4, b, _STATS_W), lambda bb, cc: (0, 0, 0))

    mih, mth, mid, mtd, fod, stats = pl.pallas_call(
        _main_body,
        grid=grid,
        in_specs=[img_spec, img_spec],
        out_specs=[mask_spec, mask_spec, mask_spec, mask_spec, fod_spec, stats_spec],
        out_shape=[
            jax.ShapeDtypeStruct((b, h, w), jnp.float32),
            jax.ShapeDtypeStruct((b, h, w), jnp.float32),
            jax.ShapeDtypeStruct((b, h, w), jnp.float32),
            jax.ShapeDtypeStruct((b, h, w), jnp.float32),
            jax.ShapeDtypeStruct((b, 4, n_stripes, h, 128), jnp.float32),
            jax.ShapeDtypeStruct((4, b, _STATS_W), jnp.float32),
        ],
        compiler_params=pltpu.CompilerParams(
            dimension_semantics=("arbitrary", "arbitrary")
        ),
    )(inputs, targets)

    rows = 4 * b
    hist_part = _sc_hist(jnp.reshape(fod, (rows * hw,)), rows, hw)

    loss = pl.pallas_call(
        functools.partial(_loss_body, batch=b, hw=hw),
        out_shape=jax.ShapeDtypeStruct((1, 1), jnp.float32),
    )(stats, jnp.reshape(hist_part, (b, 4, _SC_LANES, _NUM_BINS)))
    return (jnp.reshape(loss, ()), mih, mth, mid, mtd)
